# baseline (speedup 1.0000x reference)
import functools

import jax
import jax.numpy as jnp
from jax import lax
from jax.experimental import pallas as pl
from jax.experimental.pallas import tpu as pltpu
from jax.experimental.pallas import tpu_sc as plsc

_LANE = 128
_CHUNK = 512
_SUB = _CHUNK // _LANE


def _gather_rows(ids2d, table, num_workers):
    n_rows, _ = ids2d.shape
    n = n_rows * _LANE
    _, d = table.shape
    per_w = n // num_workers
    n_chunks = per_w // _CHUNK
    rows_per_chunk = _CHUNK // _LANE

    mesh = plsc.VectorSubcoreMesh(core_axis_name="c", subcore_axis_name="s")

    @functools.partial(
        pl.kernel,
        out_type=jax.ShapeDtypeStruct((n, d), jnp.float32),
        mesh=mesh,
        compiler_params=pltpu.CompilerParams(use_tc_tiling_on_sc=False),
        scratch_types=[
            pltpu.VMEM((2, _SUB, _LANE), jnp.int32),
            pltpu.VMEM((2, _CHUNK, d), jnp.float32),
            pltpu.SemaphoreType.DMA,
            pltpu.SemaphoreType.DMA,
            pltpu.SemaphoreType.DMA,
            pltpu.SemaphoreType.DMA,
            pltpu.SemaphoreType.DMA,
            pltpu.SemaphoreType.DMA,
        ],
    )
    def grab(ids_hbm, tab_hbm, out_hbm, idx_v, rows_v,
             si0, si1, sg0, sg1, ss0, ss1):
        idx_sems = [si0, si1]
        gat_sems = [sg0, sg1]
        st_sems = [ss0, ss1]

        nc = jax.lax.axis_size("c")
        wid = lax.axis_index("s") * nc + lax.axis_index("c")
        id_row_base = wid * (per_w // _LANE)
        out_base = wid * per_w

        def idx_copy(j, s):
            return pltpu.make_async_copy(
                ids_hbm.at[pl.ds(id_row_base + j * rows_per_chunk,
                                 rows_per_chunk)],
                idx_v.at[s],
                idx_sems[s],
            )

        def gather_copies(j, s):
            del j
            return [
                pltpu.make_async_copy(
                    tab_hbm.at[idx_v.at[s, i]],
                    rows_v.at[s, pl.ds(i * _LANE, _LANE)],
                    gat_sems[s],
                )
                for i in range(_SUB)
            ]

        def store_copy(j, s):
            return pltpu.make_async_copy(
                rows_v.at[s],
                out_hbm.at[pl.ds(out_base + j * _CHUNK, _CHUNK)],
                st_sems[s],
            )

        def step(j, s, first, last_idx, last_gather):
            for c in gather_copies(j, s):
                c.wait()
            if not last_idx:
                idx_copy(j + 2, s).start()
            if not first:
                store_copy(j - 1, 1 - s).wait()
            if not last_gather:
                idx_copy(j + 1, 1 - s).wait()
                for c in gather_copies(j + 1, 1 - s):
                    c.start()
            store_copy(j, s).start()

        idx_copy(0, 0).start()
        idx_copy(1, 1).start()
        idx_copy(0, 0).wait()
        for c in gather_copies(0, 0):
            c.start()

        step(0, 0, first=True, last_idx=False, last_gather=False)
        step(1, 1, first=False, last_idx=False, last_gather=False)

        @pl.loop(2, n_chunks - 2, step=2)
        def _(g):
            step(g, 0, first=False, last_idx=False, last_gather=False)
            step(g + 1, 1, first=False, last_idx=False, last_gather=False)

        step(n_chunks - 2, 0, first=False, last_idx=True, last_gather=False)
        step(n_chunks - 1, 1, first=False, last_idx=True, last_gather=True)
        store_copy(n_chunks - 1, 1).wait()

    return grab(ids2d, table)


def kernel(input_ids, word_embeddings):
    b, s = input_ids.shape
    v, d = word_embeddings.shape
    n = b * s
    num_workers = 32
    ids2d = input_ids.astype(jnp.int32).reshape(n // _LANE, _LANE)
    out = _gather_rows(ids2d, word_embeddings.astype(jnp.float32),
                       num_workers)
    return out.reshape(b, s, d)
